# baseline (speedup 1.0000x reference)
import functools

import jax
import jax.numpy as jnp
from jax import lax
from jax.experimental import pallas as pl
from jax.experimental.pallas import tpu as pltpu
from jax.experimental.pallas import tpu_sc as plsc

N = 10000
E = 320000
D = 128
C = 10
NUM_STEPS = 4

NC = 2
NS = 16
NW = NC * NS
CK = 40
CH = 250
NBUF = 5
NP = 10240
RPT = NP // NS

ROWS_TC = 2000


def _sc_agg(p, e3, zblk):
    mesh = plsc.VectorSubcoreMesh(core_axis_name="c", subcore_axis_name="s")

    @functools.partial(
        pl.kernel,
        out_type=jax.ShapeDtypeStruct((NC, NP, D), jnp.float32),
        mesh=mesh,
        scratch_types=[
            pltpu.VMEM((2, CH, CK), jnp.int32),
            [pltpu.VMEM((CK, D), jnp.float32)] * NBUF,
            pltpu.VMEM_SHARED((NP, D), jnp.float32),
            [pltpu.SemaphoreType.DMA] * NBUF,
            [pltpu.SemaphoreType.DMA] * NBUF,
        ],
        compiler_params=pltpu.CompilerParams(use_tc_tiling_on_sc=False),
    )
    def k(p_hbm, e_hbm, z_hbm, out_hbm, idx_v, rows, acc_sh, gsem, ssem):
        c = lax.axis_index("c")
        s = lax.axis_index("s")
        wid = c * NS + s
        pltpu.async_copy(e_hbm.at[wid], idx_v, gsem[0])
        pltpu.async_copy(z_hbm, acc_sh.at[pl.ds(s * RPT, RPT)], gsem[1])
        pltpu.make_async_copy(e_hbm.at[wid], idx_v, gsem[0]).wait()
        pltpu.make_async_copy(
            z_hbm, acc_sh.at[pl.ds(s * RPT, RPT)], gsem[1]).wait()
        plsc.subcore_barrier()

        gather = lambda i, b: pltpu.async_copy(
            p_hbm.at[idx_v.at[0, i]], rows[b], gsem[b])
        gather_wait = lambda i, b: pltpu.make_async_copy(
            p_hbm.at[idx_v.at[0, i]], rows[b], gsem[b]).wait()
        scatter = lambda i, b: pltpu.async_copy(
            rows[b], acc_sh.at[idx_v.at[1, i]], ssem[b], add=True)
        scatter_wait = lambda i, b: pltpu.make_async_copy(
            rows[b], acc_sh.at[idx_v.at[1, i]], ssem[b]).wait()

        gather(0, 0)
        gather(1, 1)
        for b in range(NBUF):
            gather_wait(b, b)
            scatter(b, b)
            if b >= 3:
                scatter_wait(b - 3, (b + 2) % NBUF)
            gather(b + 2, (b + 2) % NBUF)

        def group(g, carry):
            for b in range(NBUF):
                i = g * NBUF + b
                gather_wait(i, b)
                scatter(i, b)
                scatter_wait(i - 3, (b + 2) % NBUF)
                gather(i + 2, (b + 2) % NBUF)
            return carry

        G = CH // NBUF
        lax.fori_loop(1, G - 1, group, 0)
        for b in range(NBUF):
            i = (G - 1) * NBUF + b
            gather_wait(i, b)
            scatter(i, b)
            scatter_wait(i - 3, (b + 2) % NBUF)
            if b < 3:
                gather(i + 2, (b + 2) % NBUF)
        for j in range(CH - 3, CH):
            scatter_wait(j, j % NBUF)

        plsc.subcore_barrier()
        pltpu.sync_copy(acc_sh.at[pl.ds(s * RPT, RPT)],
                        out_hbm.at[c, pl.ds(s * RPT, RPT)])

    return k(p, e3, zblk)


def _tc_in(x, W1, b1, W2, b2, Wm, bm):
    def body(x_ref, w1, bb1, w2, bb2, wm, bbm, h_ref, p_ref):
        t = jnp.maximum(x_ref[...] @ w1[...] + bb1[...], 0.0)
        h = t @ w2[...] + bb2[...]
        h_ref[...] = h
        p_ref[...] = jnp.maximum(h @ wm[...] + bbm[...], 0.0)

    full = lambda shape: pl.BlockSpec(shape, lambda i: (0, 0))
    rows = pl.BlockSpec((ROWS_TC, D), lambda i: (i, 0))
    return pl.pallas_call(
        body,
        grid=(N // ROWS_TC,),
        in_specs=[rows, full((D, 2 * D)), full((1, 2 * D)), full((2 * D, D)),
                  full((1, D)), full((D, D)), full((1, D))],
        out_specs=[rows, rows],
        out_shape=[jax.ShapeDtypeStruct((N, D), jnp.float32),
                   jax.ShapeDtypeStruct((N, D), jnp.float32)],
    )(x, W1, b1, W2, b2, Wm, bm)


def _tc_upd(h, aggs, W1h, W1a, b1, W2, b2, g, b, Wp, bp, last):
    def body(h_ref, a0, a1, w1h, w1a, bb1, w2, bb2, gg, bb, wp, bbp,
             hn_ref, p_ref):
        h_blk = h_ref[...]
        agg = a0[...] + a1[...]
        t = jnp.maximum(h_blk @ w1h[...] + agg @ w1a[...] + bb1[...], 0.0)
        z = h_blk + t @ w2[...] + bb2[...]
        mu = jnp.mean(z, axis=-1, keepdims=True)
        zc = z - mu
        var = jnp.mean(zc * zc, axis=-1, keepdims=True)
        hn = zc * lax.rsqrt(var + 1e-5) * gg[...] + bb[...]
        hn_ref[...] = hn
        p_ref[...] = (hn @ wp[...] + bbp[...] if last
                      else jnp.maximum(hn @ wp[...] + bbp[...], 0.0))

    pdim = C if last else D
    full = lambda shape: pl.BlockSpec(shape, lambda i: (0, 0))
    rows = pl.BlockSpec((ROWS_TC, D), lambda i: (i, 0))
    prows = pl.BlockSpec((ROWS_TC, pdim), lambda i: (i, 0))
    return pl.pallas_call(
        body,
        grid=(N // ROWS_TC,),
        in_specs=[rows, rows, rows, full((D, 4 * D)), full((D, 4 * D)),
                  full((1, 4 * D)), full((4 * D, D)), full((1, D)),
                  full((1, D)), full((1, D)), full((D, pdim)),
                  full((1, pdim))],
        out_specs=[rows, prows],
        out_shape=[jax.ShapeDtypeStruct((N, D), jnp.float32),
                   jax.ShapeDtypeStruct((N, pdim), jnp.float32)],
    )(h, aggs[0], aggs[1], W1h, W1a, b1, W2, b2, g, b, Wp, bp)


def kernel(x, edge_index, W_in1, b_in1, W_in2, b_in2, W_msg, b_msg,
           W_c1, b_c1, W_c2, b_c2, ln_g, ln_b, W_out, b_out):
    src3 = edge_index[0].reshape(NW, 1, CH, CK)
    dst3 = edge_index[1].reshape(NW, 1, CH, CK)
    e3 = jnp.concatenate([src3, dst3], axis=1)
    zblk = jnp.zeros((RPT, D), jnp.float32)

    r1 = lambda v: v.reshape(1, -1)
    h, p = _tc_in(x, W_in1, r1(b_in1), W_in2, r1(b_in2), W_msg, r1(b_msg))
    W1h = jax.lax.slice_in_dim(W_c1, 0, D, axis=0)
    W1a = jax.lax.slice_in_dim(W_c1, D, 2 * D, axis=0)
    for step in range(NUM_STEPS):
        aggs = _sc_agg(p, e3, zblk)
        last = step == NUM_STEPS - 1
        Wp, bp = (W_out, b_out) if last else (W_msg, b_msg)
        h, p = _tc_upd(h, aggs, W1h, W1a, r1(b_c1), W_c2, r1(b_c2),
                       r1(ln_g), r1(ln_b), Wp, r1(bp), last)
    return p


# speedup vs baseline: 1.3406x; 1.3406x over previous
import functools

import jax
import jax.numpy as jnp
from jax import lax
from jax.experimental import pallas as pl
from jax.experimental.pallas import tpu as pltpu
from jax.experimental.pallas import tpu_sc as plsc

N = 10000
E = 320000
D = 128
C = 10
NUM_STEPS = 4

NC = 2
NS = 16
NW = NC * NS
CK = 40
CH = 250
NBUF = 5
NP = 10240
RPT = NP // NS

ROWS_TC = 2000


def _sc_agg(p, e3, zblk):
    mesh = plsc.VectorSubcoreMesh(core_axis_name="c", subcore_axis_name="s")

    @functools.partial(
        pl.kernel,
        out_type=jax.ShapeDtypeStruct((NC, NP, D), jnp.float32),
        mesh=mesh,
        scratch_types=[
            pltpu.VMEM((2, CH, CK), jnp.int32),
            [pltpu.VMEM((CK, D), jnp.float32)] * NBUF,
            pltpu.VMEM_SHARED((NP, D), jnp.float32),
            [pltpu.SemaphoreType.DMA] * NBUF,
        ],
        compiler_params=pltpu.CompilerParams(use_tc_tiling_on_sc=False),
    )
    def k(p_hbm, e_hbm, z_hbm, out_hbm, idx_v, rows, acc_sh, gsem):
        c = lax.axis_index("c")
        s = lax.axis_index("s")
        wid = c * NS + s
        pltpu.async_copy(e_hbm.at[wid], idx_v, gsem[0])
        pltpu.async_copy(z_hbm, acc_sh.at[pl.ds(s * RPT, RPT)], gsem[1])
        pltpu.make_async_copy(e_hbm.at[wid], idx_v, gsem[0]).wait()
        pltpu.make_async_copy(
            z_hbm, acc_sh.at[pl.ds(s * RPT, RPT)], gsem[1]).wait()
        plsc.subcore_barrier()

        gather = lambda i, b: pltpu.async_copy(
            p_hbm.at[idx_v.at[0, i]], rows[b], gsem[b])
        gather_wait = lambda i, b: pltpu.make_async_copy(
            p_hbm.at[idx_v.at[0, i]], rows[b], gsem[b]).wait()
        scatter = lambda i, b: pltpu.sync_copy(
            rows[b], acc_sh.at[idx_v.at[1, i]], add=True)

        for b in range(NBUF):
            gather(b, b)

        def group(g, carry):
            for b in range(NBUF):
                i = g * NBUF + b
                gather_wait(i, b)
                scatter(i, b)
                gather(i + NBUF, b)
            return carry

        G = CH // NBUF
        lax.fori_loop(0, G - 1, group, 0)
        for b in range(NBUF):
            i = (G - 1) * NBUF + b
            gather_wait(i, b)
            scatter(i, b)

        plsc.subcore_barrier()
        pltpu.sync_copy(acc_sh.at[pl.ds(s * RPT, RPT)],
                        out_hbm.at[c, pl.ds(s * RPT, RPT)])

    return k(p, e3, zblk)


def _tc_in(x, W1, b1, W2, b2, Wm, bm):
    def body(x_ref, w1, bb1, w2, bb2, wm, bbm, h_ref, p_ref):
        t = jnp.maximum(x_ref[...] @ w1[...] + bb1[...], 0.0)
        h = t @ w2[...] + bb2[...]
        h_ref[...] = h
        p_ref[...] = jnp.maximum(h @ wm[...] + bbm[...], 0.0)

    full = lambda shape: pl.BlockSpec(shape, lambda i: (0, 0))
    rows = pl.BlockSpec((ROWS_TC, D), lambda i: (i, 0))
    return pl.pallas_call(
        body,
        grid=(N // ROWS_TC,),
        in_specs=[rows, full((D, 2 * D)), full((1, 2 * D)), full((2 * D, D)),
                  full((1, D)), full((D, D)), full((1, D))],
        out_specs=[rows, rows],
        out_shape=[jax.ShapeDtypeStruct((N, D), jnp.float32),
                   jax.ShapeDtypeStruct((N, D), jnp.float32)],
    )(x, W1, b1, W2, b2, Wm, bm)


def _tc_upd(h, aggs, W1h, W1a, b1, W2, b2, g, b, Wp, bp, last):
    def body(h_ref, a0, a1, w1h, w1a, bb1, w2, bb2, gg, bb, wp, bbp,
             hn_ref, p_ref):
        h_blk = h_ref[...]
        agg = a0[...] + a1[...]
        t = jnp.maximum(h_blk @ w1h[...] + agg @ w1a[...] + bb1[...], 0.0)
        z = h_blk + t @ w2[...] + bb2[...]
        mu = jnp.mean(z, axis=-1, keepdims=True)
        zc = z - mu
        var = jnp.mean(zc * zc, axis=-1, keepdims=True)
        hn = zc * lax.rsqrt(var + 1e-5) * gg[...] + bb[...]
        hn_ref[...] = hn
        p_ref[...] = (hn @ wp[...] + bbp[...] if last
                      else jnp.maximum(hn @ wp[...] + bbp[...], 0.0))

    pdim = C if last else D
    full = lambda shape: pl.BlockSpec(shape, lambda i: (0, 0))
    rows = pl.BlockSpec((ROWS_TC, D), lambda i: (i, 0))
    prows = pl.BlockSpec((ROWS_TC, pdim), lambda i: (i, 0))
    return pl.pallas_call(
        body,
        grid=(N // ROWS_TC,),
        in_specs=[rows, rows, rows, full((D, 4 * D)), full((D, 4 * D)),
                  full((1, 4 * D)), full((4 * D, D)), full((1, D)),
                  full((1, D)), full((1, D)), full((D, pdim)),
                  full((1, pdim))],
        out_specs=[rows, prows],
        out_shape=[jax.ShapeDtypeStruct((N, D), jnp.float32),
                   jax.ShapeDtypeStruct((N, pdim), jnp.float32)],
    )(h, aggs[0], aggs[1], W1h, W1a, b1, W2, b2, g, b, Wp, bp)


def kernel(x, edge_index, W_in1, b_in1, W_in2, b_in2, W_msg, b_msg,
           W_c1, b_c1, W_c2, b_c2, ln_g, ln_b, W_out, b_out):
    src3 = edge_index[0].reshape(NW, 1, CH, CK)
    dst3 = edge_index[1].reshape(NW, 1, CH, CK)
    e3 = jnp.concatenate([src3, dst3], axis=1)
    zblk = jnp.zeros((RPT, D), jnp.float32)

    r1 = lambda v: v.reshape(1, -1)
    h, p = _tc_in(x, W_in1, r1(b_in1), W_in2, r1(b_in2), W_msg, r1(b_msg))
    W1h = jax.lax.slice_in_dim(W_c1, 0, D, axis=0)
    W1a = jax.lax.slice_in_dim(W_c1, D, 2 * D, axis=0)
    for step in range(NUM_STEPS):
        aggs = _sc_agg(p, e3, zblk)
        last = step == NUM_STEPS - 1
        Wp, bp = (W_out, b_out) if last else (W_msg, b_msg)
        h, p = _tc_upd(h, aggs, W1h, W1a, r1(b_c1), W_c2, r1(b_c2),
                       r1(ln_g), r1(ln_b), Wp, r1(bp), last)
    return p
